# Initial kernel scaffold; baseline (speedup 1.0000x reference)
import jax
import jax.numpy as jnp
from jax import lax
from jax.experimental import pallas as pl
from jax.experimental.pallas import tpu as pltpu
from jax.experimental.pallas import tpu_sc as plsc

_N0, _N1, _B, _DIN, _DH, _DOUT = 131072, 32768, 64, 128, 128, 16

_NS = 16
_RPT = _N0 // _NS
_NVEC = _RPT // 16
_RANGE = _N1 // 4
_DUMP = _RANGE
_SLAB = _RANGE // _NS
_CH = 128
_PAD = _RPT + 128


def _p1_body(x_hbm, par_hbm, out_hbm,
             pvals, idxflat, relflat, idxchunk, rel2d, rowbuf, zbuf, acc):
    c = lax.axis_index("c")
    s = lax.axis_index("s")
    base = s * _RPT

    def _zfill(i, carry):
        r = i // 8
        k = (i % 8) * 16
        zbuf[r, pl.ds(k, 16)] = jnp.zeros((16,), jnp.float32)
        return carry
    lax.fori_loop(0, _CH * 8, _zfill, 0)

    pltpu.sync_copy(par_hbm.at[pl.ds(base, _RPT)], pvals)

    for p in range(2):
        lo = (c * 2 + p) * _RANGE

        for t in range(_SLAB // _CH):
            pltpu.sync_copy(zbuf, acc.at[pl.ds(s * _SLAB + t * _CH, _CH)])
        plsc.subcore_barrier()

        def _build(i, off):
            v = pvals[pl.ds(i * 16, 16)]
            m = (v >= lo) & (v < lo + _RANGE)
            rows = base + i * 16 + lax.iota(jnp.int32, 16)
            plsc.store_compressed(idxflat.at[pl.ds(off, 16)], rows, mask=m)
            plsc.store_compressed(relflat.at[pl.ds(off, 16)], v - lo, mask=m)
            return off + jnp.sum(m.astype(jnp.int32))
        n = lax.fori_loop(0, _NVEC, _build, jnp.int32(0))

        a = (n // 16) * 16
        for k in range(8):
            pos0 = a + k * 16
            lanes = pos0 + lax.iota(jnp.int32, 16)
            keep = lanes < n
            oi = idxflat[pl.ds(pos0, 16)]
            orr = relflat[pl.ds(pos0, 16)]
            idxflat[pl.ds(pos0, 16)] = jnp.where(keep, oi, 0)
            relflat[pl.ds(pos0, 16)] = jnp.where(keep, orr, _DUMP)

        nch = (n + _CH - 1) // _CH

        def _chunk(j, carry):
            pltpu.sync_copy(idxflat.at[pl.ds(j * _CH, _CH)], idxchunk)
            pltpu.sync_copy(relflat.at[pl.ds(j * _CH, _CH)], rel2d.at[j])
            pltpu.sync_copy(x_hbm.at[idxchunk], rowbuf)
            pltpu.sync_copy(rowbuf, acc.at[rel2d.at[j]], add=True)
            return carry
        lax.fori_loop(0, nch, _chunk, 0)

        plsc.subcore_barrier()
        pltpu.sync_copy(acc.at[pl.ds(s * _SLAB, _SLAB)],
                        out_hbm.at[pl.ds(lo + s * _SLAB, _SLAB)])


_sc_p1 = pl.kernel(
    _p1_body,
    out_type=jax.ShapeDtypeStruct((_N1, _DIN), jnp.float32),
    mesh=plsc.VectorSubcoreMesh(core_axis_name="c", subcore_axis_name="s"),
    scratch_types=[
        pltpu.VMEM((_RPT,), jnp.int32),
        pltpu.VMEM((_PAD,), jnp.int32),
        pltpu.VMEM((_PAD,), jnp.int32),
        pltpu.VMEM((_CH,), jnp.int32),
        pltpu.VMEM((_PAD // _CH, _CH), jnp.int32),
        pltpu.VMEM((_CH, _DIN), jnp.float32),
        pltpu.VMEM((_CH, _DIN), jnp.float32),
        pltpu.VMEM_SHARED((_RANGE + 8, _DIN), jnp.float32),
    ],
)

_RB = 2048


def _g0p_body(seg_ref, x_ref, out_ref):
    i = pl.program_id(0)
    seg = seg_ref[...]
    ioh = lax.broadcasted_iota(jnp.int32, (_B, _RB), 0)
    oh = (ioh == seg[None, :]).astype(jnp.float32)
    con = jnp.dot(oh, x_ref[...], preferred_element_type=jnp.float32)

    @pl.when(i == 0)
    def _():
        out_ref[...] = jnp.zeros((_B, _DIN), jnp.float32)

    out_ref[...] += con


_g0p = pl.pallas_call(
    _g0p_body,
    grid=(_N0 // _RB,),
    in_specs=[pl.BlockSpec((_RB,), lambda i: (i,)),
              pl.BlockSpec((_RB, _DIN), lambda i: (i, 0))],
    out_specs=pl.BlockSpec((_B, _DIN), lambda i: (0, 0)),
    out_shape=jax.ShapeDtypeStruct((_B, _DIN), jnp.float32),
)

_MB = 2048
_NB = _N1 // _MB


def _bn_rows(z, g, b):
    m = jnp.mean(z, axis=0, keepdims=True)
    v = jnp.maximum(jnp.mean(z * z, axis=0, keepdims=True) - m * m, 0.0)
    return g[None, :] * (z - m) / jnp.sqrt(v + 1e-5) + b[None, :]


def _mlp_body(p1_ref, g0p_ref, seg1_ref, par2_ref,
              W1a_ref, b1a_ref, g1m_ref, be1m_ref, W1b_ref, b1b_ref,
              g1_ref, be1_ref,
              W2a_ref, b2a_ref, g2m_ref, be2m_ref, W2b_ref, b2b_ref,
              g2_ref, be2_ref, Wp_ref, bp_ref,
              out_ref, Z, s1, q1, s2, q2, g1p_acc, p2_acc):
    p = pl.program_id(0)
    j = pl.program_id(1)

    @pl.when((p == 0) & (j == 0))
    def _():
        s1[...] = jnp.zeros((1, _DH), jnp.float32)
        q1[...] = jnp.zeros((1, _DH), jnp.float32)
        s2[...] = jnp.zeros((1, _DH), jnp.float32)
        q2[...] = jnp.zeros((1, _DH), jnp.float32)
        g1p_acc[...] = jnp.zeros((_B, _DH), jnp.float32)
        p2_acc[...] = jnp.zeros((_B, _DH), jnp.float32)

    @pl.when(p == 0)
    def _():
        z = jnp.dot(p1_ref[...], W1a_ref[...],
                    preferred_element_type=jnp.float32) + b1a_ref[...][None, :]
        Z[pl.ds(j * _MB, _MB), :] = z
        s1[...] += jnp.sum(z, axis=0, keepdims=True)
        q1[...] += jnp.sum(z * z, axis=0, keepdims=True)

    @pl.when(p == 1)
    def _():
        m = s1[...] / _N1
        v = jnp.maximum(q1[...] / _N1 - m * m, 0.0)
        z = Z[pl.ds(j * _MB, _MB), :]
        h = jnp.maximum(
            g1m_ref[...][None, :] * (z - m) / jnp.sqrt(v + 1e-5)
            + be1m_ref[...][None, :], 0.0)
        y = jnp.dot(h, W1b_ref[...],
                    preferred_element_type=jnp.float32) + b1b_ref[...][None, :]
        Z[pl.ds(j * _MB, _MB), :] = y
        s2[...] += jnp.sum(y, axis=0, keepdims=True)
        q2[...] += jnp.sum(y * y, axis=0, keepdims=True)

    @pl.when(p == 2)
    def _():
        m = s2[...] / _N1
        v = jnp.maximum(q2[...] / _N1 - m * m, 0.0)
        y = Z[pl.ds(j * _MB, _MB), :]
        h1 = jnp.maximum(
            g1_ref[...][None, :] * (y - m) / jnp.sqrt(v + 1e-5)
            + be1_ref[...][None, :], 0.0)
        ioh = lax.broadcasted_iota(jnp.int32, (_B, _MB), 0)
        oh1 = (ioh == seg1_ref[...][None, :]).astype(jnp.float32)
        oh2 = (ioh == par2_ref[...][None, :]).astype(jnp.float32)
        g1p_acc[...] += jnp.dot(oh1, h1, preferred_element_type=jnp.float32)
        p2_acc[...] += jnp.dot(oh2, h1, preferred_element_type=jnp.float32)

        @pl.when(j == _NB - 1)
        def _():
            z2 = jnp.dot(p2_acc[...], W2a_ref[...],
                         preferred_element_type=jnp.float32) + b2a_ref[...][None, :]
            h = jnp.maximum(_bn_rows(z2, g2m_ref[...], be2m_ref[...]), 0.0)
            y2 = jnp.dot(h, W2b_ref[...],
                         preferred_element_type=jnp.float32) + b2b_ref[...][None, :]
            h2 = jnp.maximum(_bn_rows(y2, g2_ref[...], be2_ref[...]), 0.0)
            logits = (jnp.dot(g0p_ref[...], Wp_ref[0:_DIN, :],
                              preferred_element_type=jnp.float32)
                      + jnp.dot(g1p_acc[...], Wp_ref[_DIN:_DIN + _DH, :],
                                preferred_element_type=jnp.float32)
                      + jnp.dot(h2, Wp_ref[_DIN + _DH:, :],
                                preferred_element_type=jnp.float32)
                      + bp_ref[...][None, :])
            mx = jnp.max(logits, axis=1, keepdims=True)
            e = jnp.exp(logits - mx)
            out_ref[...] = e / jnp.sum(e, axis=1, keepdims=True)


def _full(shape):
    nd = len(shape)
    return pl.BlockSpec(shape, lambda p, j, _nd=nd: (0,) * _nd)


_mlp = pl.pallas_call(
    _mlp_body,
    grid=(3, _NB),
    in_specs=[
        pl.BlockSpec((_MB, _DIN), lambda p, j: (jnp.where(p == 0, j, 0), 0)),
        _full((_B, _DIN)),
        pl.BlockSpec((_MB,), lambda p, j: (jnp.where(p == 2, j, 0),)),
        pl.BlockSpec((_MB,), lambda p, j: (jnp.where(p == 2, j, 0),)),
        _full((_DIN, _DH)), _full((_DH,)), _full((_DH,)), _full((_DH,)),
        _full((_DH, _DH)), _full((_DH,)), _full((_DH,)), _full((_DH,)),
        _full((_DH, _DH)), _full((_DH,)), _full((_DH,)), _full((_DH,)),
        _full((_DH, _DH)), _full((_DH,)), _full((_DH,)), _full((_DH,)),
        _full((_DIN + 2 * _DH, _DOUT)), _full((_DOUT,)),
    ],
    out_specs=pl.BlockSpec((_B, _DOUT), lambda p, j: (0, 0)),
    out_shape=jax.ShapeDtypeStruct((_B, _DOUT), jnp.float32),
    scratch_shapes=[
        pltpu.VMEM((_N1, _DH), jnp.float32),
        pltpu.VMEM((1, _DH), jnp.float32),
        pltpu.VMEM((1, _DH), jnp.float32),
        pltpu.VMEM((1, _DH), jnp.float32),
        pltpu.VMEM((1, _DH), jnp.float32),
        pltpu.VMEM((_B, _DH), jnp.float32),
        pltpu.VMEM((_B, _DH), jnp.float32),
    ],
)


def kernel(x, W1a, b1a, g1m, be1m, W1b, b1b, g1, be1,
           W2a, b2a, g2m, be2m, W2b, b2b, g2, be2, Wp, bp,
           parent1, parent2, seg0, seg1):
    p1 = _sc_p1(x, parent1.astype(jnp.int32))
    g0p = _g0p(seg0.astype(jnp.int32), x)
    return _mlp(p1, g0p, seg1.astype(jnp.int32), parent2.astype(jnp.int32),
                W1a, b1a, g1m, be1m, W1b, b1b, g1, be1,
                W2a, b2a, g2m, be2m, W2b, b2b, g2, be2, Wp, bp)


# speedup vs baseline: 2.8882x; 2.8882x over previous
import jax
import jax.numpy as jnp
from jax import lax
from jax.experimental import pallas as pl
from jax.experimental.pallas import tpu as pltpu
from jax.experimental.pallas import tpu_sc as plsc

_N0, _N1, _B, _DIN, _DH, _DOUT = 131072, 32768, 64, 128, 128, 16

_NS = 16
_RPT = _N0 // _NS
_NVEC = _RPT // 16
_RANGE = _N1 // 4
_DUMP = _RANGE
_SLAB = _RANGE // _NS
_CH = 128
_PAD = _RPT + 128


def _p1_body(x_hbm, par_hbm, out_hbm,
             pvals, idx2d, rel2d, rowbuf, zbuf, acc):
    c = lax.axis_index("c")
    s = lax.axis_index("s")
    base = s * _RPT

    def _zfill(i, carry):
        r = i // 8
        k = (i % 8) * 16
        zbuf[r, pl.ds(k, 16)] = jnp.zeros((16,), jnp.float32)
        return carry
    lax.fori_loop(0, _CH * 8, _zfill, 0)

    pltpu.sync_copy(par_hbm.at[pl.ds(base, _RPT)], pvals)

    for p in range(2):
        lo = (c * 2 + p) * _RANGE

        for t in range(_SLAB // _CH):
            pltpu.sync_copy(zbuf, acc.at[pl.ds(s * _SLAB + t * _CH, _CH)])
        plsc.subcore_barrier()

        def _build(i, off):
            v = pvals[pl.ds(i * 16, 16)]
            m = (v >= lo) & (v < lo + _RANGE)
            cs = plsc.cumsum(m.astype(jnp.int32))
            pos = off + cs - 1
            rows = base + i * 16 + lax.iota(jnp.int32, 16)
            plsc.store_scatter(idx2d, [pos // _CH, pos % _CH], rows, mask=m)
            plsc.store_scatter(rel2d, [pos // _CH, pos % _CH], v - lo, mask=m)
            return off + jnp.sum(m.astype(jnp.int32))
        n = lax.fori_loop(0, _NVEC, _build, jnp.int32(0))

        a = (n // 16) * 16
        zero16 = jnp.zeros((16,), jnp.int32)
        dump16 = jnp.full((16,), _DUMP, jnp.int32)
        for k in range(8):
            lanes = a + k * 16 + lax.iota(jnp.int32, 16)
            padm = lanes >= n
            plsc.store_scatter(idx2d, [lanes // _CH, lanes % _CH], zero16,
                               mask=padm)
            plsc.store_scatter(rel2d, [lanes // _CH, lanes % _CH], dump16,
                               mask=padm)

        nch = (n + _CH - 1) // _CH

        def _chunk(j, carry):
            pltpu.sync_copy(x_hbm.at[idx2d.at[j]], rowbuf)
            pltpu.sync_copy(rowbuf, acc.at[rel2d.at[j]], add=True)
            return carry
        lax.fori_loop(0, nch, _chunk, 0)

        plsc.subcore_barrier()
        pltpu.sync_copy(acc.at[pl.ds(s * _SLAB, _SLAB)],
                        out_hbm.at[pl.ds(lo + s * _SLAB, _SLAB)])


_sc_p1_cache = []


def _sc_p1(x, parent1):
    if not _sc_p1_cache:
        _sc_p1_cache.append(_build_sc_p1())
    return _sc_p1_cache[0](x, parent1)


def _build_sc_p1():
    return pl.kernel(
        _p1_body,
        out_type=jax.ShapeDtypeStruct((_N1, _DIN), jnp.float32),
        mesh=plsc.VectorSubcoreMesh(core_axis_name="c", subcore_axis_name="s"),
        compiler_params=pltpu.CompilerParams(needs_layout_passes=False),
        scratch_types=[
        pltpu.VMEM((_RPT,), jnp.int32),
        pltpu.VMEM((_PAD // _CH, _CH), jnp.int32),
        pltpu.VMEM((_PAD // _CH, _CH), jnp.int32),
        pltpu.VMEM((_CH, _DIN), jnp.float32),
        pltpu.VMEM((_CH, _DIN), jnp.float32),
        pltpu.VMEM_SHARED((_RANGE + 8, _DIN), jnp.float32),
        ],
    )

_RB = 2048


def _g0p_body(seg_ref, x_ref, out_ref):
    i = pl.program_id(0)
    seg = seg_ref[...]
    ioh = lax.broadcasted_iota(jnp.int32, (_B, _RB), 0)
    oh = (ioh == seg[None, :]).astype(jnp.float32)
    con = jnp.dot(oh, x_ref[...], preferred_element_type=jnp.float32,
                    precision=lax.Precision.HIGHEST)

    @pl.when(i == 0)
    def _():
        out_ref[...] = jnp.zeros((_B, _DIN), jnp.float32)

    out_ref[...] += con


_g0p = pl.pallas_call(
    _g0p_body,
    grid=(_N0 // _RB,),
    in_specs=[pl.BlockSpec((_RB,), lambda i: (i,)),
              pl.BlockSpec((_RB, _DIN), lambda i: (i, 0))],
    out_specs=pl.BlockSpec((_B, _DIN), lambda i: (0, 0)),
    out_shape=jax.ShapeDtypeStruct((_B, _DIN), jnp.float32),
)

_MB = 2048
_NB = _N1 // _MB


def _bn_rows(z, g, b):
    m = jnp.mean(z, axis=0, keepdims=True)
    v = jnp.maximum(jnp.mean(z * z, axis=0, keepdims=True) - m * m, 0.0)
    return g[None, :] * (z - m) / jnp.sqrt(v + 1e-5) + b[None, :]


def _mlp_body(p1_ref, g0p_ref, seg1_ref, par2_ref,
              W1a_ref, b1a_ref, g1m_ref, be1m_ref, W1b_ref, b1b_ref,
              g1_ref, be1_ref,
              W2a_ref, b2a_ref, g2m_ref, be2m_ref, W2b_ref, b2b_ref,
              g2_ref, be2_ref, Wp_ref, bp_ref,
              out_ref, Z, s1, q1, s2, q2, g1p_acc, p2_acc):
    p = pl.program_id(0)
    j = pl.program_id(1)

    @pl.when((p == 0) & (j == 0))
    def _():
        s1[...] = jnp.zeros((1, _DH), jnp.float32)
        q1[...] = jnp.zeros((1, _DH), jnp.float32)
        s2[...] = jnp.zeros((1, _DH), jnp.float32)
        q2[...] = jnp.zeros((1, _DH), jnp.float32)
        g1p_acc[...] = jnp.zeros((_B, _DH), jnp.float32)
        p2_acc[...] = jnp.zeros((_B, _DH), jnp.float32)

    @pl.when(p == 0)
    def _():
        z = jnp.dot(p1_ref[...], W1a_ref[...],
                    preferred_element_type=jnp.float32,
                    precision=lax.Precision.HIGHEST) + b1a_ref[...][None, :]
        Z[pl.ds(j * _MB, _MB), :] = z
        s1[...] += jnp.sum(z, axis=0, keepdims=True)
        q1[...] += jnp.sum(z * z, axis=0, keepdims=True)

    @pl.when(p == 1)
    def _():
        m = s1[...] / _N1
        v = jnp.maximum(q1[...] / _N1 - m * m, 0.0)
        z = Z[pl.ds(j * _MB, _MB), :]
        h = jnp.maximum(
            g1m_ref[...][None, :] * (z - m) / jnp.sqrt(v + 1e-5)
            + be1m_ref[...][None, :], 0.0)
        y = jnp.dot(h, W1b_ref[...],
                    preferred_element_type=jnp.float32,
                    precision=lax.Precision.HIGHEST) + b1b_ref[...][None, :]
        Z[pl.ds(j * _MB, _MB), :] = y
        s2[...] += jnp.sum(y, axis=0, keepdims=True)
        q2[...] += jnp.sum(y * y, axis=0, keepdims=True)

    @pl.when(p == 2)
    def _():
        m = s2[...] / _N1
        v = jnp.maximum(q2[...] / _N1 - m * m, 0.0)
        y = Z[pl.ds(j * _MB, _MB), :]
        h1 = jnp.maximum(
            g1_ref[...][None, :] * (y - m) / jnp.sqrt(v + 1e-5)
            + be1_ref[...][None, :], 0.0)
        ioh = lax.broadcasted_iota(jnp.int32, (_B, _MB), 0)
        oh1 = (ioh == seg1_ref[...][None, :]).astype(jnp.float32)
        oh2 = (ioh == par2_ref[...][None, :]).astype(jnp.float32)
        g1p_acc[...] += jnp.dot(oh1, h1, preferred_element_type=jnp.float32,
                    precision=lax.Precision.HIGHEST)
        p2_acc[...] += jnp.dot(oh2, h1, preferred_element_type=jnp.float32,
                    precision=lax.Precision.HIGHEST)

        @pl.when(j == _NB - 1)
        def _():
            z2 = jnp.dot(p2_acc[...], W2a_ref[...],
                         preferred_element_type=jnp.float32,
                    precision=lax.Precision.HIGHEST) + b2a_ref[...][None, :]
            h = jnp.maximum(_bn_rows(z2, g2m_ref[...], be2m_ref[...]), 0.0)
            y2 = jnp.dot(h, W2b_ref[...],
                         preferred_element_type=jnp.float32,
                    precision=lax.Precision.HIGHEST) + b2b_ref[...][None, :]
            h2 = jnp.maximum(_bn_rows(y2, g2_ref[...], be2_ref[...]), 0.0)
            logits = (jnp.dot(g0p_ref[...], Wp_ref[0:_DIN, :],
                              preferred_element_type=jnp.float32,
                    precision=lax.Precision.HIGHEST)
                      + jnp.dot(g1p_acc[...], Wp_ref[_DIN:_DIN + _DH, :],
                                preferred_element_type=jnp.float32,
                    precision=lax.Precision.HIGHEST)
                      + jnp.dot(h2, Wp_ref[_DIN + _DH:, :],
                                preferred_element_type=jnp.float32,
                    precision=lax.Precision.HIGHEST)
                      + bp_ref[...][None, :])
            mx = jnp.max(logits, axis=1, keepdims=True)
            e = jnp.exp(logits - mx)
            out_ref[...] = e / jnp.sum(e, axis=1, keepdims=True)


def _full(shape):
    nd = len(shape)
    return pl.BlockSpec(shape, lambda p, j, _nd=nd: (0,) * _nd)


_mlp = pl.pallas_call(
    _mlp_body,
    grid=(3, _NB),
    in_specs=[
        pl.BlockSpec((_MB, _DIN), lambda p, j: (jnp.where(p == 0, j, 0), 0)),
        _full((_B, _DIN)),
        pl.BlockSpec((_MB,), lambda p, j: (jnp.where(p == 2, j, 0),)),
        pl.BlockSpec((_MB,), lambda p, j: (jnp.where(p == 2, j, 0),)),
        _full((_DIN, _DH)), _full((_DH,)), _full((_DH,)), _full((_DH,)),
        _full((_DH, _DH)), _full((_DH,)), _full((_DH,)), _full((_DH,)),
        _full((_DH, _DH)), _full((_DH,)), _full((_DH,)), _full((_DH,)),
        _full((_DH, _DH)), _full((_DH,)), _full((_DH,)), _full((_DH,)),
        _full((_DIN + 2 * _DH, _DOUT)), _full((_DOUT,)),
    ],
    out_specs=pl.BlockSpec((_B, _DOUT), lambda p, j: (0, 0)),
    out_shape=jax.ShapeDtypeStruct((_B, _DOUT), jnp.float32),
    scratch_shapes=[
        pltpu.VMEM((_N1, _DH), jnp.float32),
        pltpu.VMEM((1, _DH), jnp.float32),
        pltpu.VMEM((1, _DH), jnp.float32),
        pltpu.VMEM((1, _DH), jnp.float32),
        pltpu.VMEM((1, _DH), jnp.float32),
        pltpu.VMEM((_B, _DH), jnp.float32),
        pltpu.VMEM((_B, _DH), jnp.float32),
    ],
)


def kernel(x, W1a, b1a, g1m, be1m, W1b, b1b, g1, be1,
           W2a, b2a, g2m, be2m, W2b, b2b, g2, be2, Wp, bp,
           parent1, parent2, seg0, seg1):
    p1 = _sc_p1(x, parent1.astype(jnp.int32))
    g0p = _g0p(seg0.astype(jnp.int32), x)
    return _mlp(p1, g0p, seg1.astype(jnp.int32), parent2.astype(jnp.int32),
                W1a, b1a, g1m, be1m, W1b, b1b, g1, be1,
                W2a, b2a, g2m, be2m, W2b, b2b, g2, be2, Wp, bp)


# speedup vs baseline: 3.1486x; 1.0902x over previous
import jax
import jax.numpy as jnp
from jax import lax
from jax.experimental import pallas as pl
from jax.experimental.pallas import tpu as pltpu
from jax.experimental.pallas import tpu_sc as plsc

_N0, _N1, _B, _DIN, _DH, _DOUT = 131072, 32768, 64, 128, 128, 16

_NS = 16
_RPT = _N0 // _NS
_NVEC = _RPT // 16
_RANGE = _N1 // 4
_DUMP = _RANGE
_SLAB = _RANGE // _NS
_CH = 128
_PAD = _RPT + 128
_NBUF = 2
_ZR = 32


def _p1_body(x_hbm, par_hbm, out_hbm,
             pvals, idx2d, rel2d, rowbuf, zbuf, acc,
             gsem0, gsem1, ssem0, ssem1):
    c = lax.axis_index("c")
    s = lax.axis_index("s")
    base = s * _RPT

    def _zfill(i, carry):
        r = i // 8
        k = (i % 8) * 16
        zbuf[r, pl.ds(k, 16)] = jnp.zeros((16,), jnp.float32)
        return carry
    lax.fori_loop(0, _ZR * 8, _zfill, 0)

    pltpu.sync_copy(par_hbm.at[pl.ds(base, _RPT)], pvals)

    for p in range(2):
        lo = (c * 2 + p) * _RANGE

        for t in range(_SLAB // _ZR):
            pltpu.sync_copy(zbuf, acc.at[pl.ds(s * _SLAB + t * _ZR, _ZR)])
        plsc.subcore_barrier()

        def _build(i, off):
            v = pvals[pl.ds(i * 16, 16)]
            m = (v >= lo) & (v < lo + _RANGE)
            cs = plsc.cumsum(m.astype(jnp.int32))
            pos = off + cs - 1
            rows = base + i * 16 + lax.iota(jnp.int32, 16)
            plsc.store_scatter(idx2d, [pos // _CH, pos % _CH], rows, mask=m)
            plsc.store_scatter(rel2d, [pos // _CH, pos % _CH], v - lo, mask=m)
            return off + jnp.sum(m.astype(jnp.int32))
        n = lax.fori_loop(0, _NVEC, _build, jnp.int32(0))

        a = (n // 16) * 16
        zero16 = jnp.zeros((16,), jnp.int32)
        dump16 = jnp.full((16,), _DUMP, jnp.int32)
        for k in range(8):
            lanes = a + k * 16 + lax.iota(jnp.int32, 16)
            padm = lanes >= n
            plsc.store_scatter(idx2d, [lanes // _CH, lanes % _CH], zero16,
                               mask=padm)
            plsc.store_scatter(rel2d, [lanes // _CH, lanes % _CH], dump16,
                               mask=padm)

        nch = (n + _CH - 1) // _CH

        ngrp = (nch + _NBUF - 1) // _NBUF
        gsems = (gsem0, gsem1)
        ssems = (ssem0, ssem1)

        def _group(g, carry):
            for b in range(_NBUF):
                j = g * _NBUF + b

                @pl.when(j < nch)
                def _():
                    @pl.when(j >= _NBUF)
                    def _():
                        pltpu.make_async_copy(rowbuf.at[b],
                                              acc.at[rel2d.at[j]],
                                              ssems[b]).wait()
                    pltpu.async_copy(x_hbm.at[idx2d.at[j]], rowbuf.at[b],
                                     gsems[b])
            for b in range(_NBUF):
                j = g * _NBUF + b

                @pl.when(j < nch)
                def _():
                    pltpu.make_async_copy(x_hbm.at[idx2d.at[j]],
                                          rowbuf.at[b], gsems[b]).wait()
                    pltpu.async_copy(rowbuf.at[b], acc.at[rel2d.at[j]],
                                     ssems[b], add=True)
            return carry
        lax.fori_loop(0, ngrp, _group, 0)
        for b in range(_NBUF):
            @pl.when(b < nch)
            def _():
                pltpu.make_async_copy(rowbuf.at[b], acc.at[rel2d.at[0]],
                                      ssems[b]).wait()

        plsc.subcore_barrier()
        pltpu.sync_copy(acc.at[pl.ds(s * _SLAB, _SLAB)],
                        out_hbm.at[pl.ds(lo + s * _SLAB, _SLAB)])


_sc_p1_cache = []


def _sc_p1(x, parent1):
    if not _sc_p1_cache:
        _sc_p1_cache.append(_build_sc_p1())
    return _sc_p1_cache[0](x, parent1)


def _build_sc_p1():
    return pl.kernel(
        _p1_body,
        out_type=jax.ShapeDtypeStruct((_N1, _DIN), jnp.float32),
        mesh=plsc.VectorSubcoreMesh(core_axis_name="c", subcore_axis_name="s"),
        compiler_params=pltpu.CompilerParams(needs_layout_passes=False),
        scratch_types=[
        pltpu.VMEM((_RPT,), jnp.int32),
        pltpu.VMEM((_PAD // _CH, _CH), jnp.int32),
        pltpu.VMEM((_PAD // _CH, _CH), jnp.int32),
        pltpu.VMEM((_NBUF, _CH, _DIN), jnp.float32),
        pltpu.VMEM((_ZR, _DIN), jnp.float32),
        pltpu.VMEM_SHARED((_RANGE + 8, _DIN), jnp.float32),
        pltpu.SemaphoreType.DMA,
        pltpu.SemaphoreType.DMA,
        pltpu.SemaphoreType.DMA,
        pltpu.SemaphoreType.DMA,
        ],
    )

_RB = 2048


def _g0p_body(seg_ref, x_ref, out_ref):
    i = pl.program_id(0)
    seg = seg_ref[...]
    ioh = lax.broadcasted_iota(jnp.int32, (_B, _RB), 0)
    oh = (ioh == seg[None, :]).astype(jnp.float32)
    con = jnp.dot(oh, x_ref[...], preferred_element_type=jnp.float32,
                    precision=lax.Precision.HIGHEST)

    @pl.when(i == 0)
    def _():
        out_ref[...] = jnp.zeros((_B, _DIN), jnp.float32)

    out_ref[...] += con


_g0p = pl.pallas_call(
    _g0p_body,
    grid=(_N0 // _RB,),
    in_specs=[pl.BlockSpec((_RB,), lambda i: (i,)),
              pl.BlockSpec((_RB, _DIN), lambda i: (i, 0))],
    out_specs=pl.BlockSpec((_B, _DIN), lambda i: (0, 0)),
    out_shape=jax.ShapeDtypeStruct((_B, _DIN), jnp.float32),
)

_MB = 2048
_NB = _N1 // _MB


def _bn_rows(z, g, b):
    m = jnp.mean(z, axis=0, keepdims=True)
    v = jnp.maximum(jnp.mean(z * z, axis=0, keepdims=True) - m * m, 0.0)
    return g[None, :] * (z - m) / jnp.sqrt(v + 1e-5) + b[None, :]


def _mlp_body(p1_ref, g0p_ref, seg1_ref, par2_ref,
              W1a_ref, b1a_ref, g1m_ref, be1m_ref, W1b_ref, b1b_ref,
              g1_ref, be1_ref,
              W2a_ref, b2a_ref, g2m_ref, be2m_ref, W2b_ref, b2b_ref,
              g2_ref, be2_ref, Wp_ref, bp_ref,
              out_ref, Z, s1, q1, s2, q2, g1p_acc, p2_acc):
    p = pl.program_id(0)
    j = pl.program_id(1)

    @pl.when((p == 0) & (j == 0))
    def _():
        s1[...] = jnp.zeros((1, _DH), jnp.float32)
        q1[...] = jnp.zeros((1, _DH), jnp.float32)
        s2[...] = jnp.zeros((1, _DH), jnp.float32)
        q2[...] = jnp.zeros((1, _DH), jnp.float32)
        g1p_acc[...] = jnp.zeros((_B, _DH), jnp.float32)
        p2_acc[...] = jnp.zeros((_B, _DH), jnp.float32)

    @pl.when(p == 0)
    def _():
        z = jnp.dot(p1_ref[...], W1a_ref[...],
                    preferred_element_type=jnp.float32) + b1a_ref[...][None, :]
        Z[pl.ds(j * _MB, _MB), :] = z
        s1[...] += jnp.sum(z, axis=0, keepdims=True)
        q1[...] += jnp.sum(z * z, axis=0, keepdims=True)

    @pl.when(p == 1)
    def _():
        m = s1[...] / _N1
        v = jnp.maximum(q1[...] / _N1 - m * m, 0.0)
        z = Z[pl.ds(j * _MB, _MB), :]
        h = jnp.maximum(
            g1m_ref[...][None, :] * (z - m) / jnp.sqrt(v + 1e-5)
            + be1m_ref[...][None, :], 0.0)
        y = jnp.dot(h, W1b_ref[...],
                    preferred_element_type=jnp.float32) + b1b_ref[...][None, :]
        Z[pl.ds(j * _MB, _MB), :] = y
        s2[...] += jnp.sum(y, axis=0, keepdims=True)
        q2[...] += jnp.sum(y * y, axis=0, keepdims=True)

    @pl.when(p == 2)
    def _():
        m = s2[...] / _N1
        v = jnp.maximum(q2[...] / _N1 - m * m, 0.0)
        y = Z[pl.ds(j * _MB, _MB), :]
        h1 = jnp.maximum(
            g1_ref[...][None, :] * (y - m) / jnp.sqrt(v + 1e-5)
            + be1_ref[...][None, :], 0.0)
        ioh = lax.broadcasted_iota(jnp.int32, (_B, _MB), 0)
        oh1 = (ioh == seg1_ref[...][None, :]).astype(jnp.float32)
        oh2 = (ioh == par2_ref[...][None, :]).astype(jnp.float32)
        g1p_acc[...] += jnp.dot(oh1, h1, preferred_element_type=jnp.float32,
                    precision=lax.Precision.HIGHEST)
        p2_acc[...] += jnp.dot(oh2, h1, preferred_element_type=jnp.float32,
                    precision=lax.Precision.HIGHEST)

        @pl.when(j == _NB - 1)
        def _():
            z2 = jnp.dot(p2_acc[...], W2a_ref[...],
                         preferred_element_type=jnp.float32) + b2a_ref[...][None, :]
            h = jnp.maximum(_bn_rows(z2, g2m_ref[...], be2m_ref[...]), 0.0)
            y2 = jnp.dot(h, W2b_ref[...],
                         preferred_element_type=jnp.float32) + b2b_ref[...][None, :]
            h2 = jnp.maximum(_bn_rows(y2, g2_ref[...], be2_ref[...]), 0.0)
            logits = (jnp.dot(g0p_ref[...], Wp_ref[0:_DIN, :],
                              preferred_element_type=jnp.float32)
                      + jnp.dot(g1p_acc[...], Wp_ref[_DIN:_DIN + _DH, :],
                                preferred_element_type=jnp.float32)
                      + jnp.dot(h2, Wp_ref[_DIN + _DH:, :],
                                preferred_element_type=jnp.float32)
                      + bp_ref[...][None, :])
            mx = jnp.max(logits, axis=1, keepdims=True)
            e = jnp.exp(logits - mx)
            out_ref[...] = e / jnp.sum(e, axis=1, keepdims=True)


def _full(shape):
    nd = len(shape)
    return pl.BlockSpec(shape, lambda p, j, _nd=nd: (0,) * _nd)


_mlp = pl.pallas_call(
    _mlp_body,
    grid=(3, _NB),
    in_specs=[
        pl.BlockSpec((_MB, _DIN), lambda p, j: (jnp.where(p == 0, j, 0), 0)),
        _full((_B, _DIN)),
        pl.BlockSpec((_MB,), lambda p, j: (jnp.where(p == 2, j, 0),)),
        pl.BlockSpec((_MB,), lambda p, j: (jnp.where(p == 2, j, 0),)),
        _full((_DIN, _DH)), _full((_DH,)), _full((_DH,)), _full((_DH,)),
        _full((_DH, _DH)), _full((_DH,)), _full((_DH,)), _full((_DH,)),
        _full((_DH, _DH)), _full((_DH,)), _full((_DH,)), _full((_DH,)),
        _full((_DH, _DH)), _full((_DH,)), _full((_DH,)), _full((_DH,)),
        _full((_DIN + 2 * _DH, _DOUT)), _full((_DOUT,)),
    ],
    out_specs=pl.BlockSpec((_B, _DOUT), lambda p, j: (0, 0)),
    out_shape=jax.ShapeDtypeStruct((_B, _DOUT), jnp.float32),
    scratch_shapes=[
        pltpu.VMEM((_N1, _DH), jnp.float32),
        pltpu.VMEM((1, _DH), jnp.float32),
        pltpu.VMEM((1, _DH), jnp.float32),
        pltpu.VMEM((1, _DH), jnp.float32),
        pltpu.VMEM((1, _DH), jnp.float32),
        pltpu.VMEM((_B, _DH), jnp.float32),
        pltpu.VMEM((_B, _DH), jnp.float32),
    ],
)


def kernel(x, W1a, b1a, g1m, be1m, W1b, b1b, g1, be1,
           W2a, b2a, g2m, be2m, W2b, b2b, g2, be2, Wp, bp,
           parent1, parent2, seg0, seg1):
    p1 = _sc_p1(x, parent1.astype(jnp.int32))
    g0p = _g0p(seg0.astype(jnp.int32), x)
    return _mlp(p1, g0p, seg1.astype(jnp.int32), parent2.astype(jnp.int32),
                W1a, b1a, g1m, be1m, W1b, b1b, g1, be1,
                W2a, b2a, g2m, be2m, W2b, b2b, g2, be2, Wp, bp)
